# initial kernel scaffold (unmeasured)
import jax
import jax.numpy as jnp
from jax import lax
from jax.experimental import pallas as pl
from jax.experimental.pallas import tpu as pltpu

N_DEV = 8


def kernel(x, w_mat):
    m_per, k = x.shape
    _, n_per = w_mat.shape

    x = x.astype(jnp.bfloat16)
    w = w_mat.astype(jnp.bfloat16)

    def body(x_ref, w_ref, out_ref, comm_ref, send_sems, recv_sems):
        my_pos = lax.axis_index("i")
        left = (my_pos + N_DEV - 1) % N_DEV
        right = (my_pos + 1) % N_DEV

        barrier_sem = pltpu.get_barrier_semaphore()
        for nbr in [left, right]:
            pl.semaphore_signal(
                barrier_sem, inc=1,
                device_id=(nbr,), device_id_type=pl.DeviceIdType.MESH,
            )
        pl.semaphore_wait(barrier_sem, 2)

        comm_ref[0, :, :] = x_ref[:, :]

        def gemm_for_slot(slot, hop):
            origin = (my_pos + N_DEV - hop) % N_DEV
            acc = jnp.dot(
                comm_ref[slot, :, :], w_ref[:, :],
                preferred_element_type=jnp.float32,
            )
            out_ref[pl.ds(origin * m_per, m_per), :] = jnp.maximum(acc, 0.0)

        for h in range(N_DEV - 1):
            send_slot = h % 2
            recv_slot = (h + 1) % 2
            rdma = pltpu.make_async_remote_copy(
                src_ref=comm_ref.at[send_slot],
                dst_ref=comm_ref.at[recv_slot],
                send_sem=send_sems.at[h],
                recv_sem=recv_sems.at[h],
                device_id=(right,),
                device_id_type=pl.DeviceIdType.MESH,
            )
            rdma.start()
            gemm_for_slot(send_slot, h)
            rdma.wait()

        gemm_for_slot((N_DEV - 1) % 2, N_DEV - 1)

    return pl.pallas_call(
        body,
        out_shape=jax.ShapeDtypeStruct((N_DEV * m_per, n_per), jnp.float32),
        in_specs=[
            pl.BlockSpec(memory_space=pltpu.VMEM),
            pl.BlockSpec(memory_space=pltpu.VMEM),
        ],
        out_specs=pl.BlockSpec(memory_space=pltpu.VMEM),
        scratch_shapes=[
            pltpu.VMEM((2, m_per, k), jnp.bfloat16),
            pltpu.SemaphoreType.DMA((N_DEV - 1,)),
            pltpu.SemaphoreType.DMA((N_DEV - 1,)),
        ],
        compiler_params=pltpu.CompilerParams(
            collective_id=0,
            vmem_limit_bytes=128 * 1024 * 1024,
        ),
    )(x, w)


# baseline (device time: 1338150 ns/iter reference)
import jax
import jax.numpy as jnp
from jax import lax
from jax.experimental import pallas as pl
from jax.experimental.pallas import tpu as pltpu

N_DEV = 8


def kernel(x, w_mat):
    m_per, k = x.shape
    _, n_per = w_mat.shape

    x = x.astype(jnp.bfloat16)
    w = w_mat.astype(jnp.bfloat16)

    def body(x_ref, w_ref, out_ref, comm_ref, send_sems, recv_sems, local_sem):
        my_pos = lax.axis_index("i")
        left = (my_pos + N_DEV - 1) % N_DEV
        right = (my_pos + 1) % N_DEV

        load = pltpu.make_async_copy(x_ref, comm_ref.at[0], local_sem)
        load.start()

        barrier_sem = pltpu.get_barrier_semaphore()
        for nbr in [left, right]:
            pl.semaphore_signal(
                barrier_sem, inc=1,
                device_id=(nbr,), device_id_type=pl.DeviceIdType.MESH,
            )
        pl.semaphore_wait(barrier_sem, 2)
        load.wait()

        def gemm_for_slot(slot, hop):
            origin = (my_pos + N_DEV - hop) % N_DEV
            acc = jnp.dot(
                comm_ref[slot, :, :], w_ref[:, :],
                preferred_element_type=jnp.float32,
            )
            out_ref[pl.ds(origin * m_per, m_per), :] = jnp.maximum(acc, 0.0)

        def hop_body(h, carry):
            send_slot = h % 2
            recv_slot = 1 - send_slot
            rdma = pltpu.make_async_remote_copy(
                src_ref=comm_ref.at[send_slot],
                dst_ref=comm_ref.at[recv_slot],
                send_sem=send_sems.at[h],
                recv_sem=recv_sems.at[h],
                device_id=(right,),
                device_id_type=pl.DeviceIdType.MESH,
            )
            rdma.start()
            gemm_for_slot(send_slot, h)
            rdma.wait()
            return carry

        lax.fori_loop(0, N_DEV - 1, hop_body, 0)

        gemm_for_slot((N_DEV - 1) % 2, N_DEV - 1)

    return pl.pallas_call(
        body,
        out_shape=jax.ShapeDtypeStruct((N_DEV * m_per, n_per), jnp.float32),
        in_specs=[
            pl.BlockSpec(memory_space=pl.ANY),
            pl.BlockSpec(memory_space=pltpu.VMEM),
        ],
        out_specs=pl.BlockSpec(memory_space=pltpu.VMEM),
        scratch_shapes=[
            pltpu.VMEM((2, m_per, k), jnp.bfloat16),
            pltpu.SemaphoreType.DMA((N_DEV - 1,)),
            pltpu.SemaphoreType.DMA((N_DEV - 1,)),
            pltpu.SemaphoreType.DMA,
        ],
        compiler_params=pltpu.CompilerParams(
            collective_id=0,
            vmem_limit_bytes=128 * 1024 * 1024,
        ),
    )(x, w)


# device time: 709172 ns/iter; 1.8869x vs baseline; 1.8869x over previous
import jax
import jax.numpy as jnp
from jax import lax
from jax.experimental import pallas as pl
from jax.experimental.pallas import tpu as pltpu

N_DEV = 8


def _ring_next(l):
    return jnp.where(
        l == 3, 7, jnp.where(l == 4, 0, jnp.where(l < 3, l + 1, l - 1))
    )


def _ring_prev(l):
    return jnp.where(
        l == 7, 3, jnp.where(l == 0, 4, jnp.where(l <= 3, l - 1, l + 1))
    )


def kernel(x, w_mat):
    m_per, k = x.shape
    _, n_per = w_mat.shape
    m_half = m_per // 2

    x = x.astype(jnp.bfloat16)
    w = w_mat.astype(jnp.bfloat16)

    def body(x_ref, w_ref, out_ref, cw_ref, ccw_ref,
             cw_send, cw_recv, ccw_send, ccw_recv, load_sems):
        my_pos = lax.axis_index("i")
        cw_nbr = _ring_next(my_pos)
        ccw_nbr = _ring_prev(my_pos)

        load_top = pltpu.make_async_copy(
            x_ref.at[pl.ds(0, m_half)], cw_ref.at[0], load_sems.at[0])
        load_bot = pltpu.make_async_copy(
            x_ref.at[pl.ds(m_half, m_half)], ccw_ref.at[0], load_sems.at[1])
        load_top.start()
        load_bot.start()

        barrier_sem = pltpu.get_barrier_semaphore()
        for nbr in [cw_nbr, ccw_nbr]:
            pl.semaphore_signal(
                barrier_sem, inc=1,
                device_id=(nbr,), device_id_type=pl.DeviceIdType.MESH,
            )
        pl.semaphore_wait(barrier_sem, 2)
        load_top.wait()
        load_bot.wait()

        def gemm_half(comm_ref, slot, origin, half):
            acc = jnp.dot(
                comm_ref[slot, :, :], w_ref[:, :],
                preferred_element_type=jnp.float32,
            )
            row = origin * m_per + half * m_half
            out_ref[pl.ds(row, m_half), :] = jnp.maximum(acc, 0.0)

        def hop_body(h, carry):
            cw_origin, ccw_origin = carry
            send_slot = h % 2
            recv_slot = 1 - send_slot
            cw_rdma = pltpu.make_async_remote_copy(
                src_ref=cw_ref.at[send_slot],
                dst_ref=cw_ref.at[recv_slot],
                send_sem=cw_send.at[h],
                recv_sem=cw_recv.at[h],
                device_id=(cw_nbr,),
                device_id_type=pl.DeviceIdType.MESH,
            )
            ccw_rdma = pltpu.make_async_remote_copy(
                src_ref=ccw_ref.at[send_slot],
                dst_ref=ccw_ref.at[recv_slot],
                send_sem=ccw_send.at[h],
                recv_sem=ccw_recv.at[h],
                device_id=(ccw_nbr,),
                device_id_type=pl.DeviceIdType.MESH,
            )
            cw_rdma.start()
            ccw_rdma.start()
            gemm_half(cw_ref, send_slot, cw_origin, 0)
            gemm_half(ccw_ref, send_slot, ccw_origin, 1)
            cw_rdma.wait()
            ccw_rdma.wait()
            return _ring_prev(cw_origin), _ring_next(ccw_origin)

        cw_origin, ccw_origin = lax.fori_loop(
            0, N_DEV - 1, hop_body, (my_pos, my_pos))

        last = (N_DEV - 1) % 2
        gemm_half(cw_ref, last, cw_origin, 0)
        gemm_half(ccw_ref, last, ccw_origin, 1)

    return pl.pallas_call(
        body,
        out_shape=jax.ShapeDtypeStruct((N_DEV * m_per, n_per), jnp.float32),
        in_specs=[
            pl.BlockSpec(memory_space=pl.ANY),
            pl.BlockSpec(memory_space=pltpu.VMEM),
        ],
        out_specs=pl.BlockSpec(memory_space=pltpu.VMEM),
        scratch_shapes=[
            pltpu.VMEM((2, m_half, k), jnp.bfloat16),
            pltpu.VMEM((2, m_half, k), jnp.bfloat16),
            pltpu.SemaphoreType.DMA((N_DEV - 1,)),
            pltpu.SemaphoreType.DMA((N_DEV - 1,)),
            pltpu.SemaphoreType.DMA((N_DEV - 1,)),
            pltpu.SemaphoreType.DMA((N_DEV - 1,)),
            pltpu.SemaphoreType.DMA((2,)),
        ],
        compiler_params=pltpu.CompilerParams(
            collective_id=0,
            vmem_limit_bytes=128 * 1024 * 1024,
        ),
    )(x, w)


# device time: 682656 ns/iter; 1.9602x vs baseline; 1.0388x over previous
import jax
import jax.numpy as jnp
from jax import lax
from jax.experimental import pallas as pl
from jax.experimental.pallas import tpu as pltpu

N_DEV = 8
N_HOP = N_DEV - 1
X_PIECES = 16
W_PIECES = 8


def _ring_next(l):
    return jnp.where(
        l == 3, 7, jnp.where(l == 4, 0, jnp.where(l < 3, l + 1, l - 1))
    )


def _ring_prev(l):
    return jnp.where(
        l == 7, 3, jnp.where(l == 0, 4, jnp.where(l <= 3, l - 1, l + 1))
    )


def kernel(x, w_mat):
    m_per, k = x.shape
    _, n_per = w_mat.shape
    m_half = m_per // 2
    x_rows = m_per // X_PIECES
    w_rows = k // W_PIECES

    def body(x_hbm, w_hbm, out_hbm, cw_ref, ccw_ref, w_bf, x_tmp, w_tmp,
             acc_ref, cw_send, cw_recv, ccw_send, ccw_recv,
             xload_sems, wload_sems, out_sems):
        my_pos = lax.axis_index("i")
        cw_nbr = _ring_next(my_pos)
        ccw_nbr = _ring_prev(my_pos)

        barrier_sem = pltpu.get_barrier_semaphore()
        for nbr in [cw_nbr, ccw_nbr]:
            pl.semaphore_signal(
                barrier_sem, inc=1,
                device_id=(nbr,), device_id_type=pl.DeviceIdType.MESH,
            )

        def x_load(p, slot):
            return pltpu.make_async_copy(
                x_hbm.at[pl.ds(p * x_rows, x_rows)],
                x_tmp.at[slot], xload_sems.at[slot])

        x_load(0, 0).start()
        for p in range(X_PIECES):
            if p + 1 < X_PIECES:
                x_load(p + 1, (p + 1) % 2).start()
            x_load(p, p % 2).wait()
            piece = x_tmp[p % 2].astype(jnp.bfloat16)
            rows = p * x_rows
            if rows < m_half:
                cw_ref[0, pl.ds(rows, x_rows), :] = piece
            else:
                ccw_ref[0, pl.ds(rows - m_half, x_rows), :] = piece

        pl.semaphore_wait(barrier_sem, 2)

        def gemm_half(comm_ref, slot, origin, half):
            acc_ref[half, :, :] = jnp.maximum(
                jnp.dot(comm_ref[slot, :, :], w_bf[:, :],
                        preferred_element_type=jnp.float32),
                0.0,
            )
            out_copy(origin, half).start()

        def out_copy(origin, half):
            row = origin * m_per + half * m_half
            return pltpu.make_async_copy(
                acc_ref.at[half],
                out_hbm.at[pl.ds(row, m_half), :],
                out_sems.at[half],
            )

        def hop_body(h, carry):
            cw_origin, ccw_origin = carry
            send_slot = h % 2
            recv_slot = 1 - send_slot
            cw_rdma = pltpu.make_async_remote_copy(
                src_ref=cw_ref.at[send_slot],
                dst_ref=cw_ref.at[recv_slot],
                send_sem=cw_send.at[h],
                recv_sem=cw_recv.at[h],
                device_id=(cw_nbr,),
                device_id_type=pl.DeviceIdType.MESH,
            )
            ccw_rdma = pltpu.make_async_remote_copy(
                src_ref=ccw_ref.at[send_slot],
                dst_ref=ccw_ref.at[recv_slot],
                send_sem=ccw_send.at[h],
                recv_sem=ccw_recv.at[h],
                device_id=(ccw_nbr,),
                device_id_type=pl.DeviceIdType.MESH,
            )
            cw_rdma.start()
            ccw_rdma.start()

            @pl.when(h == 0)
            def _():
                for p in range(W_PIECES):
                    pltpu.make_async_copy(
                        w_hbm.at[pl.ds(p * w_rows, w_rows)],
                        w_tmp.at[p % 2], wload_sems.at[p % 2],
                    ).start()
                    if p >= 1:
                        pltpu.make_async_copy(
                            w_hbm.at[pl.ds((p - 1) * w_rows, w_rows)],
                            w_tmp.at[(p - 1) % 2], wload_sems.at[(p - 1) % 2],
                        ).wait()
                        w_bf[pl.ds((p - 1) * w_rows, w_rows), :] = (
                            w_tmp[(p - 1) % 2].astype(jnp.bfloat16))
                pltpu.make_async_copy(
                    w_hbm.at[pl.ds((W_PIECES - 1) * w_rows, w_rows)],
                    w_tmp.at[(W_PIECES - 1) % 2],
                    wload_sems.at[(W_PIECES - 1) % 2],
                ).wait()
                w_bf[pl.ds((W_PIECES - 1) * w_rows, w_rows), :] = (
                    w_tmp[(W_PIECES - 1) % 2].astype(jnp.bfloat16))

            @pl.when(h > 0)
            def _():
                out_copy(_ring_next(cw_origin), 0).wait()
                out_copy(_ring_prev(ccw_origin), 1).wait()

            gemm_half(cw_ref, send_slot, cw_origin, 0)
            gemm_half(ccw_ref, send_slot, ccw_origin, 1)
            cw_rdma.wait()
            ccw_rdma.wait()
            return _ring_prev(cw_origin), _ring_next(ccw_origin)

        cw_origin, ccw_origin = lax.fori_loop(
            0, N_HOP, hop_body, (my_pos, my_pos))

        last = N_HOP % 2
        out_copy(_ring_next(cw_origin), 0).wait()
        out_copy(_ring_prev(ccw_origin), 1).wait()
        gemm_half(cw_ref, last, cw_origin, 0)
        gemm_half(ccw_ref, last, ccw_origin, 1)
        out_copy(cw_origin, 0).wait()
        out_copy(ccw_origin, 1).wait()

    return pl.pallas_call(
        body,
        out_shape=jax.ShapeDtypeStruct((N_DEV * m_per, n_per), jnp.float32),
        in_specs=[
            pl.BlockSpec(memory_space=pl.ANY),
            pl.BlockSpec(memory_space=pl.ANY),
        ],
        out_specs=pl.BlockSpec(memory_space=pl.ANY),
        scratch_shapes=[
            pltpu.VMEM((2, m_half, k), jnp.bfloat16),
            pltpu.VMEM((2, m_half, k), jnp.bfloat16),
            pltpu.VMEM((k, n_per), jnp.bfloat16),
            pltpu.VMEM((2, x_rows, k), jnp.float32),
            pltpu.VMEM((2, w_rows, n_per), jnp.float32),
            pltpu.VMEM((2, m_half, n_per), jnp.float32),
            pltpu.SemaphoreType.DMA((N_HOP,)),
            pltpu.SemaphoreType.DMA((N_HOP,)),
            pltpu.SemaphoreType.DMA((N_HOP,)),
            pltpu.SemaphoreType.DMA((N_HOP,)),
            pltpu.SemaphoreType.DMA((2,)),
            pltpu.SemaphoreType.DMA((2,)),
            pltpu.SemaphoreType.DMA((2,)),
        ],
        compiler_params=pltpu.CompilerParams(
            collective_id=0,
            vmem_limit_bytes=128 * 1024 * 1024,
        ),
    )(x, w_mat)


# device time: 672095 ns/iter; 1.9910x vs baseline; 1.0157x over previous
import jax
import jax.numpy as jnp
from jax import lax
from jax.experimental import pallas as pl
from jax.experimental.pallas import tpu as pltpu

N_DEV = 8
N_HOP = N_DEV - 1
X_PIECES = 16
N_SUB = 4
W_PIECES = 8


def _ring_next(l):
    return jnp.where(
        l == 3, 7, jnp.where(l == 4, 0, jnp.where(l < 3, l + 1, l - 1))
    )


def _ring_prev(l):
    return jnp.where(
        l == 7, 3, jnp.where(l == 0, 4, jnp.where(l <= 3, l - 1, l + 1))
    )


def kernel(x, w_mat):
    m_per, k = x.shape
    _, n_per = w_mat.shape
    m_half = m_per // 2
    x_rows = m_per // X_PIECES
    sub_rows = m_half // N_SUB
    w_rows = k // W_PIECES

    def body(x_hbm, w_hbm, out_hbm, cw_ref, ccw_ref, w_bf, x_tmp, w_tmp,
             acc_ref, cw_send, cw_recv, ccw_send, ccw_recv,
             sub_send, sub_recv, xload_sems, wload_sems, out_sems):
        my_pos = lax.axis_index("i")
        cw_nbr = _ring_next(my_pos)
        ccw_nbr = _ring_prev(my_pos)

        barrier_sem = pltpu.get_barrier_semaphore()
        for nbr in [cw_nbr, ccw_nbr]:
            pl.semaphore_signal(
                barrier_sem, inc=1,
                device_id=(nbr,), device_id_type=pl.DeviceIdType.MESH,
            )

        pieces_per_sub = m_half // N_SUB // x_rows
        order = []
        for s in range(N_SUB):
            order += list(range(s * pieces_per_sub, (s + 1) * pieces_per_sub))
            order += [X_PIECES // 2 + i for i in
                      range(s * pieces_per_sub, (s + 1) * pieces_per_sub)]

        def x_load(p, slot):
            return pltpu.make_async_copy(
                x_hbm.at[pl.ds(p * x_rows, x_rows)],
                x_tmp.at[slot], xload_sems.at[slot])

        def sub_rdma(dir_idx, s):
            comm = cw_ref if dir_idx == 0 else ccw_ref
            nbr = cw_nbr if dir_idx == 0 else ccw_nbr
            return pltpu.make_async_remote_copy(
                src_ref=comm.at[0, pl.ds(s * sub_rows, sub_rows)],
                dst_ref=comm.at[1, pl.ds(s * sub_rows, sub_rows)],
                send_sem=sub_send.at[dir_idx, s],
                recv_sem=sub_recv.at[dir_idx, s],
                device_id=(nbr,),
                device_id_type=pl.DeviceIdType.MESH,
            )

        x_load(order[0], 0).start()
        barrier_waited = False
        for i, p in enumerate(order):
            if i + 1 < X_PIECES:
                x_load(order[i + 1], (i + 1) % 2).start()
            x_load(p, i % 2).wait()
            piece = x_tmp[i % 2].astype(jnp.bfloat16)
            rows = p * x_rows
            if rows < m_half:
                cw_ref[0, pl.ds(rows, x_rows), :] = piece
            else:
                ccw_ref[0, pl.ds(rows - m_half, x_rows), :] = piece
            done = i + 1
            group = 2 * pieces_per_sub
            if done % group == pieces_per_sub:
                if not barrier_waited:
                    pl.semaphore_wait(barrier_sem, 2)
                    barrier_waited = True
                sub_rdma(0, done // group).start()
            elif done % group == 0:
                sub_rdma(1, done // group - 1).start()

        for p in range(W_PIECES):
            pltpu.make_async_copy(
                w_hbm.at[pl.ds(p * w_rows, w_rows)],
                w_tmp.at[p % 2], wload_sems.at[p % 2],
            ).start()
            if p >= 1:
                pltpu.make_async_copy(
                    w_hbm.at[pl.ds((p - 1) * w_rows, w_rows)],
                    w_tmp.at[(p - 1) % 2], wload_sems.at[(p - 1) % 2],
                ).wait()
                w_bf[pl.ds((p - 1) * w_rows, w_rows), :] = (
                    w_tmp[(p - 1) % 2].astype(jnp.bfloat16))
        pltpu.make_async_copy(
            w_hbm.at[pl.ds((W_PIECES - 1) * w_rows, w_rows)],
            w_tmp.at[(W_PIECES - 1) % 2],
            wload_sems.at[(W_PIECES - 1) % 2],
        ).wait()
        w_bf[pl.ds((W_PIECES - 1) * w_rows, w_rows), :] = (
            w_tmp[(W_PIECES - 1) % 2].astype(jnp.bfloat16))

        def out_copy(origin, half):
            row = origin * m_per + half * m_half
            return pltpu.make_async_copy(
                acc_ref.at[half],
                out_hbm.at[pl.ds(row, m_half), :],
                out_sems.at[half],
            )

        def gemm_half(comm_ref, slot, origin, half):
            acc_ref[half, :, :] = jnp.maximum(
                jnp.dot(comm_ref[slot, :, :], w_bf[:, :],
                        preferred_element_type=jnp.float32),
                0.0,
            )
            out_copy(origin, half).start()

        gemm_half(cw_ref, 0, my_pos, 0)
        gemm_half(ccw_ref, 0, my_pos, 1)

        for s in range(N_SUB):
            sub_rdma(0, s).wait()
            sub_rdma(1, s).wait()

        def hop_body(h, carry):
            cw_origin, ccw_origin = carry
            send_slot = h % 2
            recv_slot = 1 - send_slot
            cw_rdma = pltpu.make_async_remote_copy(
                src_ref=cw_ref.at[send_slot],
                dst_ref=cw_ref.at[recv_slot],
                send_sem=cw_send.at[h],
                recv_sem=cw_recv.at[h],
                device_id=(cw_nbr,),
                device_id_type=pl.DeviceIdType.MESH,
            )
            ccw_rdma = pltpu.make_async_remote_copy(
                src_ref=ccw_ref.at[send_slot],
                dst_ref=ccw_ref.at[recv_slot],
                send_sem=ccw_send.at[h],
                recv_sem=ccw_recv.at[h],
                device_id=(ccw_nbr,),
                device_id_type=pl.DeviceIdType.MESH,
            )
            cw_rdma.start()
            ccw_rdma.start()

            out_copy(_ring_next(cw_origin), 0).wait()
            out_copy(_ring_prev(ccw_origin), 1).wait()

            gemm_half(cw_ref, send_slot, cw_origin, 0)
            gemm_half(ccw_ref, send_slot, ccw_origin, 1)
            cw_rdma.wait()
            ccw_rdma.wait()
            return _ring_prev(cw_origin), _ring_next(ccw_origin)

        cw_origin, ccw_origin = lax.fori_loop(
            1, N_HOP, hop_body,
            (_ring_prev(my_pos), _ring_next(my_pos)))

        last = N_HOP % 2
        out_copy(_ring_next(cw_origin), 0).wait()
        out_copy(_ring_prev(ccw_origin), 1).wait()
        gemm_half(cw_ref, last, cw_origin, 0)
        gemm_half(ccw_ref, last, ccw_origin, 1)
        out_copy(cw_origin, 0).wait()
        out_copy(ccw_origin, 1).wait()

    return pl.pallas_call(
        body,
        out_shape=jax.ShapeDtypeStruct((N_DEV * m_per, n_per), jnp.float32),
        in_specs=[
            pl.BlockSpec(memory_space=pl.ANY),
            pl.BlockSpec(memory_space=pl.ANY),
        ],
        out_specs=pl.BlockSpec(memory_space=pl.ANY),
        scratch_shapes=[
            pltpu.VMEM((2, m_half, k), jnp.bfloat16),
            pltpu.VMEM((2, m_half, k), jnp.bfloat16),
            pltpu.VMEM((k, n_per), jnp.bfloat16),
            pltpu.VMEM((2, x_rows, k), jnp.float32),
            pltpu.VMEM((2, w_rows, n_per), jnp.float32),
            pltpu.VMEM((2, m_half, n_per), jnp.float32),
            pltpu.SemaphoreType.DMA((N_HOP,)),
            pltpu.SemaphoreType.DMA((N_HOP,)),
            pltpu.SemaphoreType.DMA((N_HOP,)),
            pltpu.SemaphoreType.DMA((N_HOP,)),
            pltpu.SemaphoreType.DMA((2, N_SUB)),
            pltpu.SemaphoreType.DMA((2, N_SUB)),
            pltpu.SemaphoreType.DMA((2,)),
            pltpu.SemaphoreType.DMA((2,)),
            pltpu.SemaphoreType.DMA((2,)),
        ],
        compiler_params=pltpu.CompilerParams(
            collective_id=0,
            vmem_limit_bytes=128 * 1024 * 1024,
        ),
    )(x, w_mat)


# device time: 661541 ns/iter; 2.0228x vs baseline; 1.0160x over previous
import jax
import jax.numpy as jnp
from jax import lax
from jax.experimental import pallas as pl
from jax.experimental.pallas import tpu as pltpu

N_DEV = 8
N_HOP = N_DEV - 1
X_PIECES = 16
N_SUB = 4
W_PIECES = 8


def _ring_next(l):
    return jnp.where(
        l == 3, 7, jnp.where(l == 4, 0, jnp.where(l < 3, l + 1, l - 1))
    )


def _ring_prev(l):
    return jnp.where(
        l == 7, 3, jnp.where(l == 0, 4, jnp.where(l <= 3, l - 1, l + 1))
    )


def kernel(x, w_mat):
    m_per, k = x.shape
    _, n_per = w_mat.shape
    m_half = m_per // 2
    x_rows = m_per // X_PIECES
    sub_rows = m_half // N_SUB
    w_rows = k // W_PIECES

    def body(x_hbm, w_hbm, out_hbm, cw_ref, ccw_ref, w_bf, x_tmp, w_tmp,
             acc_ref, cw_send, cw_recv, ccw_send, ccw_recv,
             xload_sems, wload_sems, out_sems):
        my_pos = lax.axis_index("i")
        cw_nbr = _ring_next(my_pos)
        ccw_nbr = _ring_prev(my_pos)

        barrier_sem = pltpu.get_barrier_semaphore()
        for nbr in [cw_nbr, ccw_nbr]:
            pl.semaphore_signal(
                barrier_sem, inc=1,
                device_id=(nbr,), device_id_type=pl.DeviceIdType.MESH,
            )

        def sub_rdma(dir_idx, h, q, send_slot, recv_slot):
            comm = cw_ref if dir_idx == 0 else ccw_ref
            nbr = cw_nbr if dir_idx == 0 else ccw_nbr
            send = cw_send if dir_idx == 0 else ccw_send
            recv = cw_recv if dir_idx == 0 else ccw_recv
            return pltpu.make_async_remote_copy(
                src_ref=comm.at[send_slot, pl.ds(q * sub_rows, sub_rows)],
                dst_ref=comm.at[recv_slot, pl.ds(q * sub_rows, sub_rows)],
                send_sem=send.at[h, q],
                recv_sem=recv.at[h, q],
                device_id=(nbr,),
                device_id_type=pl.DeviceIdType.MESH,
            )

        pieces_per_sub = m_half // N_SUB // x_rows
        order = []
        for s in range(N_SUB):
            order += list(range(s * pieces_per_sub, (s + 1) * pieces_per_sub))
            order += [X_PIECES // 2 + i for i in
                      range(s * pieces_per_sub, (s + 1) * pieces_per_sub)]

        def x_load(p, slot):
            return pltpu.make_async_copy(
                x_hbm.at[pl.ds(p * x_rows, x_rows)],
                x_tmp.at[slot], xload_sems.at[slot])

        x_load(order[0], 0).start()
        barrier_waited = False
        for i, p in enumerate(order):
            if i + 1 < X_PIECES:
                x_load(order[i + 1], (i + 1) % 2).start()
            x_load(p, i % 2).wait()
            piece = x_tmp[i % 2].astype(jnp.bfloat16)
            rows = p * x_rows
            if rows < m_half:
                cw_ref[0, pl.ds(rows, x_rows), :] = piece
            else:
                ccw_ref[0, pl.ds(rows - m_half, x_rows), :] = piece
            done = i + 1
            group = 2 * pieces_per_sub
            if done % group == pieces_per_sub:
                if not barrier_waited:
                    pl.semaphore_wait(barrier_sem, 2)
                    barrier_waited = True
                sub_rdma(0, 0, done // group, 0, 1).start()
            elif done % group == 0:
                sub_rdma(1, 0, done // group - 1, 0, 1).start()

        for p in range(W_PIECES):
            pltpu.make_async_copy(
                w_hbm.at[pl.ds(p * w_rows, w_rows)],
                w_tmp.at[p % 2], wload_sems.at[p % 2],
            ).start()
            if p >= 1:
                pltpu.make_async_copy(
                    w_hbm.at[pl.ds((p - 1) * w_rows, w_rows)],
                    w_tmp.at[(p - 1) % 2], wload_sems.at[(p - 1) % 2],
                ).wait()
                w_bf[pl.ds((p - 1) * w_rows, w_rows), :] = (
                    w_tmp[(p - 1) % 2].astype(jnp.bfloat16))
        pltpu.make_async_copy(
            w_hbm.at[pl.ds((W_PIECES - 1) * w_rows, w_rows)],
            w_tmp.at[(W_PIECES - 1) % 2],
            wload_sems.at[(W_PIECES - 1) % 2],
        ).wait()
        w_bf[pl.ds((W_PIECES - 1) * w_rows, w_rows), :] = (
            w_tmp[(W_PIECES - 1) % 2].astype(jnp.bfloat16))

        def out_copy(origin, half):
            row = origin * m_per + half * m_half
            return pltpu.make_async_copy(
                acc_ref.at[half],
                out_hbm.at[pl.ds(row, m_half), :],
                out_sems.at[half],
            )

        def gemm_half(comm_ref, slot, origin, half):
            acc_ref[half, :, :] = jnp.maximum(
                jnp.dot(comm_ref[slot, :, :], w_bf[:, :],
                        preferred_element_type=jnp.float32),
                0.0,
            )
            out_copy(origin, half).start()

        gemm_half(cw_ref, 0, my_pos, 0)
        gemm_half(ccw_ref, 0, my_pos, 1)

        def hop_body(h, carry):
            cw_origin, ccw_origin = carry
            send_slot = h % 2
            recv_slot = 1 - send_slot
            for q in range(N_SUB):
                sub_rdma(0, h - 1, q, recv_slot, send_slot).wait()
                sub_rdma(0, h, q, send_slot, recv_slot).start()
                sub_rdma(1, h - 1, q, recv_slot, send_slot).wait()
                sub_rdma(1, h, q, send_slot, recv_slot).start()

            out_copy(_ring_next(cw_origin), 0).wait()
            out_copy(_ring_prev(ccw_origin), 1).wait()

            gemm_half(cw_ref, send_slot, cw_origin, 0)
            gemm_half(ccw_ref, send_slot, ccw_origin, 1)
            return _ring_prev(cw_origin), _ring_next(ccw_origin)

        cw_origin, ccw_origin = lax.fori_loop(
            1, N_HOP, hop_body,
            (_ring_prev(my_pos), _ring_next(my_pos)))

        last = (N_HOP - 1) % 2
        for q in range(N_SUB):
            sub_rdma(0, N_HOP - 1, q, last, 1 - last).wait()
            sub_rdma(1, N_HOP - 1, q, last, 1 - last).wait()
        out_copy(_ring_next(cw_origin), 0).wait()
        out_copy(_ring_prev(ccw_origin), 1).wait()
        gemm_half(cw_ref, 1 - last, cw_origin, 0)
        gemm_half(ccw_ref, 1 - last, ccw_origin, 1)
        out_copy(cw_origin, 0).wait()
        out_copy(ccw_origin, 1).wait()

    return pl.pallas_call(
        body,
        out_shape=jax.ShapeDtypeStruct((N_DEV * m_per, n_per), jnp.float32),
        in_specs=[
            pl.BlockSpec(memory_space=pl.ANY),
            pl.BlockSpec(memory_space=pl.ANY),
        ],
        out_specs=pl.BlockSpec(memory_space=pl.ANY),
        scratch_shapes=[
            pltpu.VMEM((2, m_half, k), jnp.bfloat16),
            pltpu.VMEM((2, m_half, k), jnp.bfloat16),
            pltpu.VMEM((k, n_per), jnp.bfloat16),
            pltpu.VMEM((2, x_rows, k), jnp.float32),
            pltpu.VMEM((2, w_rows, n_per), jnp.float32),
            pltpu.VMEM((2, m_half, n_per), jnp.float32),
            pltpu.SemaphoreType.DMA((N_HOP, N_SUB)),
            pltpu.SemaphoreType.DMA((N_HOP, N_SUB)),
            pltpu.SemaphoreType.DMA((N_HOP, N_SUB)),
            pltpu.SemaphoreType.DMA((N_HOP, N_SUB)),
            pltpu.SemaphoreType.DMA((2,)),
            pltpu.SemaphoreType.DMA((2,)),
            pltpu.SemaphoreType.DMA((2,)),
        ],
        compiler_params=pltpu.CompilerParams(
            collective_id=0,
            vmem_limit_bytes=128 * 1024 * 1024,
        ),
    )(x, w_mat)


# device time: 660674 ns/iter; 2.0254x vs baseline; 1.0013x over previous
import jax
import jax.numpy as jnp
from jax import lax
from jax.experimental import pallas as pl
from jax.experimental.pallas import tpu as pltpu

N_DEV = 8
N_HOP = N_DEV - 1
X_PIECES = 16
N_SUB = 8
W_PIECES = 8


def _ring_next(l):
    return jnp.where(
        l == 3, 7, jnp.where(l == 4, 0, jnp.where(l < 3, l + 1, l - 1))
    )


def _ring_prev(l):
    return jnp.where(
        l == 7, 3, jnp.where(l == 0, 4, jnp.where(l <= 3, l - 1, l + 1))
    )


def kernel(x, w_mat):
    m_per, k = x.shape
    _, n_per = w_mat.shape
    m_half = m_per // 2
    x_rows = m_per // X_PIECES
    sub_rows = m_half // N_SUB
    w_rows = k // W_PIECES

    def body(x_hbm, w_hbm, out_hbm, cw_ref, ccw_ref, w_bf, x_tmp, w_tmp,
             acc_ref, cw_send, cw_recv, ccw_send, ccw_recv,
             xload_sems, wload_sems, out_sems):
        my_pos = lax.axis_index("i")
        cw_nbr = _ring_next(my_pos)
        ccw_nbr = _ring_prev(my_pos)

        barrier_sem = pltpu.get_barrier_semaphore()
        for nbr in [cw_nbr, ccw_nbr]:
            pl.semaphore_signal(
                barrier_sem, inc=1,
                device_id=(nbr,), device_id_type=pl.DeviceIdType.MESH,
            )

        def sub_rdma(dir_idx, h, q, send_slot, recv_slot):
            comm = cw_ref if dir_idx == 0 else ccw_ref
            nbr = cw_nbr if dir_idx == 0 else ccw_nbr
            send = cw_send if dir_idx == 0 else ccw_send
            recv = cw_recv if dir_idx == 0 else ccw_recv
            return pltpu.make_async_remote_copy(
                src_ref=comm.at[send_slot, pl.ds(q * sub_rows, sub_rows)],
                dst_ref=comm.at[recv_slot, pl.ds(q * sub_rows, sub_rows)],
                send_sem=send.at[h, q],
                recv_sem=recv.at[h, q],
                device_id=(nbr,),
                device_id_type=pl.DeviceIdType.MESH,
            )

        pieces_per_sub = m_half // N_SUB // x_rows
        order = []
        for s in range(N_SUB):
            order += list(range(s * pieces_per_sub, (s + 1) * pieces_per_sub))
            order += [X_PIECES // 2 + i for i in
                      range(s * pieces_per_sub, (s + 1) * pieces_per_sub)]

        def x_load(p, slot):
            return pltpu.make_async_copy(
                x_hbm.at[pl.ds(p * x_rows, x_rows)],
                x_tmp.at[slot], xload_sems.at[slot])

        x_load(order[0], 0).start()
        barrier_waited = False
        for i, p in enumerate(order):
            if i + 1 < X_PIECES:
                x_load(order[i + 1], (i + 1) % 2).start()
            x_load(p, i % 2).wait()
            piece = x_tmp[i % 2].astype(jnp.bfloat16)
            rows = p * x_rows
            if rows < m_half:
                cw_ref[0, pl.ds(rows, x_rows), :] = piece
            else:
                ccw_ref[0, pl.ds(rows - m_half, x_rows), :] = piece
            done = i + 1
            group = 2 * pieces_per_sub
            if done % group == pieces_per_sub:
                if not barrier_waited:
                    pl.semaphore_wait(barrier_sem, 2)
                    barrier_waited = True
                sub_rdma(0, 0, done // group, 0, 1).start()
            elif done % group == 0:
                sub_rdma(1, 0, done // group - 1, 0, 1).start()

        for p in range(W_PIECES):
            pltpu.make_async_copy(
                w_hbm.at[pl.ds(p * w_rows, w_rows)],
                w_tmp.at[p % 2], wload_sems.at[p % 2],
            ).start()
            if p >= 1:
                pltpu.make_async_copy(
                    w_hbm.at[pl.ds((p - 1) * w_rows, w_rows)],
                    w_tmp.at[(p - 1) % 2], wload_sems.at[(p - 1) % 2],
                ).wait()
                w_bf[pl.ds((p - 1) * w_rows, w_rows), :] = (
                    w_tmp[(p - 1) % 2].astype(jnp.bfloat16))
        pltpu.make_async_copy(
            w_hbm.at[pl.ds((W_PIECES - 1) * w_rows, w_rows)],
            w_tmp.at[(W_PIECES - 1) % 2],
            wload_sems.at[(W_PIECES - 1) % 2],
        ).wait()
        w_bf[pl.ds((W_PIECES - 1) * w_rows, w_rows), :] = (
            w_tmp[(W_PIECES - 1) % 2].astype(jnp.bfloat16))

        def out_copy(origin, half):
            row = origin * m_per + half * m_half
            return pltpu.make_async_copy(
                acc_ref.at[half],
                out_hbm.at[pl.ds(row, m_half), :],
                out_sems.at[half],
            )

        def gemm_half(comm_ref, slot, origin, half):
            acc_ref[half, :, :] = jnp.maximum(
                jnp.dot(comm_ref[slot, :, :], w_bf[:, :],
                        preferred_element_type=jnp.float32),
                0.0,
            )
            out_copy(origin, half).start()

        gemm_half(cw_ref, 0, my_pos, 0)
        gemm_half(ccw_ref, 0, my_pos, 1)

        def hop_body(h, carry):
            cw_origin, ccw_origin = carry
            send_slot = h % 2
            recv_slot = 1 - send_slot
            for q in range(N_SUB):
                sub_rdma(0, h - 1, q, recv_slot, send_slot).wait()
                sub_rdma(0, h, q, send_slot, recv_slot).start()
                sub_rdma(1, h - 1, q, recv_slot, send_slot).wait()
                sub_rdma(1, h, q, send_slot, recv_slot).start()

            out_copy(_ring_next(cw_origin), 0).wait()
            out_copy(_ring_prev(ccw_origin), 1).wait()

            gemm_half(cw_ref, send_slot, cw_origin, 0)
            gemm_half(ccw_ref, send_slot, ccw_origin, 1)
            return _ring_prev(cw_origin), _ring_next(ccw_origin)

        cw_origin, ccw_origin = lax.fori_loop(
            1, N_HOP, hop_body,
            (_ring_prev(my_pos), _ring_next(my_pos)))

        last = (N_HOP - 1) % 2
        for q in range(N_SUB):
            sub_rdma(0, N_HOP - 1, q, last, 1 - last).wait()
            sub_rdma(1, N_HOP - 1, q, last, 1 - last).wait()
        out_copy(_ring_next(cw_origin), 0).wait()
        out_copy(_ring_prev(ccw_origin), 1).wait()
        gemm_half(cw_ref, 1 - last, cw_origin, 0)
        gemm_half(ccw_ref, 1 - last, ccw_origin, 1)
        out_copy(cw_origin, 0).wait()
        out_copy(ccw_origin, 1).wait()

    return pl.pallas_call(
        body,
        out_shape=jax.ShapeDtypeStruct((N_DEV * m_per, n_per), jnp.float32),
        in_specs=[
            pl.BlockSpec(memory_space=pl.ANY),
            pl.BlockSpec(memory_space=pl.ANY),
        ],
        out_specs=pl.BlockSpec(memory_space=pl.ANY),
        scratch_shapes=[
            pltpu.VMEM((2, m_half, k), jnp.bfloat16),
            pltpu.VMEM((2, m_half, k), jnp.bfloat16),
            pltpu.VMEM((k, n_per), jnp.bfloat16),
            pltpu.VMEM((2, x_rows, k), jnp.float32),
            pltpu.VMEM((2, w_rows, n_per), jnp.float32),
            pltpu.VMEM((2, m_half, n_per), jnp.float32),
            pltpu.SemaphoreType.DMA((N_HOP, N_SUB)),
            pltpu.SemaphoreType.DMA((N_HOP, N_SUB)),
            pltpu.SemaphoreType.DMA((N_HOP, N_SUB)),
            pltpu.SemaphoreType.DMA((N_HOP, N_SUB)),
            pltpu.SemaphoreType.DMA((2,)),
            pltpu.SemaphoreType.DMA((2,)),
            pltpu.SemaphoreType.DMA((2,)),
        ],
        compiler_params=pltpu.CompilerParams(
            collective_id=0,
            vmem_limit_bytes=128 * 1024 * 1024,
        ),
    )(x, w_mat)
